# baseline (speedup 1.0000x reference)
import functools

import jax
import jax.numpy as jnp
from jax import lax
from jax.experimental import pallas as pl
from jax.experimental.pallas import tpu as pltpu
from jax.experimental.pallas import tpu_sc as plsc

_B = 320000
_D = 128
_RPB = 128
_NB = _B // _RPB
_NC = 2
_NS = 16
_NW = _NC * _NS
_NBF = _NB // _NW
_REM = _NB - _NBF * _NW
_RING = 3
_GR = 256
_G = (_NBF * _RPB) // _GR
_TRIPS = _G // _RING
assert _G == _TRIPS * _RING

_mesh = plsc.VectorSubcoreMesh(core_axis_name="c", subcore_axis_name="s")


@functools.partial(
    pl.kernel,
    mesh=_mesh,
    out_type=jax.ShapeDtypeStruct((_B, _D), jnp.float32),
    scratch_types=[
        pltpu.VMEM(((_NBF + 1) * _RPB,), jnp.int32),
        pltpu.VMEM((_RING, _GR, _D), jnp.float32),
        pltpu.SemaphoreType.DMA,
        pltpu.SemaphoreType.DMA,
    ],
)
def _gather(x_hbm, src_hbm, out_hbm, idx_v, rows_v, gsem, wsem):
    wid = lax.axis_index("s") * _NC + lax.axis_index("c")
    b0 = _NBF * wid + jnp.minimum(wid, _REM)
    r0 = b0 * _RPB

    pltpu.sync_copy(src_hbm.at[0, pl.ds(r0, _NBF * _RPB)],
                    idx_v.at[pl.ds(0, _NBF * _RPB)])

    @pl.when(wid < _REM)
    def _():
        pltpu.sync_copy(src_hbm.at[0, pl.ds(r0 + _NBF * _RPB, _RPB)],
                        idx_v.at[pl.ds(_NBF * _RPB, _RPB)])

    def fire_gather(g, p):
        pltpu.make_async_copy(
            x_hbm.at[idx_v.at[pl.ds(g * _GR, _GR)]],
            rows_v.at[p], gsem).start()

    def drain_gather(p):
        pltpu.make_async_copy(x_hbm.at[pl.ds(0, _GR)],
                              rows_v.at[p], gsem).wait()

    def fire_store(g, p):
        pltpu.make_async_copy(
            rows_v.at[p], out_hbm.at[pl.ds(r0 + g * _GR, _GR)],
            wsem).start()

    def drain_store(p):
        pltpu.make_async_copy(
            rows_v.at[p], out_hbm.at[pl.ds(0, _GR)], wsem).wait()

    for p in range(_RING - 1):
        fire_gather(p, p)

    def ring(i, carry):
        gbase = _RING * i
        for j in range(_RING):
            p = j
            pm1 = (j - 1) % _RING
            drain_gather(p)
            if j == 0:
                @pl.when(i > 0)
                def _():
                    drain_store(pm1)
            else:
                drain_store(pm1)
            if j == 0:
                fire_gather(gbase + _RING - 1, pm1)
            else:
                @pl.when(i < _TRIPS - 1)
                def _(j=j, pm1=pm1):
                    fire_gather(gbase + _RING - 1 + j, pm1)
            fire_store(gbase + j, p)
        return carry

    lax.fori_loop(0, _TRIPS, ring, 0)
    drain_store((_G - 1) % _RING)

    @pl.when(wid < _REM)
    def _():
        pltpu.async_copy(
            x_hbm.at[idx_v.at[pl.ds(_NBF * _RPB, _RPB)]],
            rows_v.at[0, pl.ds(0, _RPB)], gsem).wait()
        pltpu.sync_copy(rows_v.at[0, pl.ds(0, _RPB)],
                        out_hbm.at[pl.ds(r0 + _NBF * _RPB, _RPB)])


def kernel(x, edge_index):
    messages = _gather(x, edge_index.astype(jnp.int32))
    return (jnp.copy(x), jnp.copy(edge_index), messages)


# speedup vs baseline: 1.0014x; 1.0014x over previous
import functools

import jax
import jax.numpy as jnp
from jax import lax
from jax.experimental import pallas as pl
from jax.experimental.pallas import tpu as pltpu
from jax.experimental.pallas import tpu_sc as plsc

_B = 320000
_D = 128
_RPB = 128
_NB = _B // _RPB
_NC = 2
_NS = 16
_NW = _NC * _NS
_NBF = _NB // _NW
_REM = _NB - _NBF * _NW
_RING = 3
_GR = 256
_G = (_NBF * _RPB) // _GR
_TRIPS = _G // _RING
assert _G == _TRIPS * _RING

_mesh = plsc.VectorSubcoreMesh(core_axis_name="c", subcore_axis_name="s")


@functools.partial(
    pl.kernel,
    mesh=_mesh,
    out_type=jax.ShapeDtypeStruct((_B, _D), jnp.float32),
    scratch_types=[
        pltpu.VMEM(((_NBF + 1) * _RPB,), jnp.int32),
        pltpu.VMEM((_RING, _GR, _D), jnp.float32),
        pltpu.SemaphoreType.DMA,
        pltpu.SemaphoreType.DMA,
    ],
)
def _gather(x_hbm, src_hbm, out_hbm, idx_v, rows_v, gsem, wsem):
    wid = lax.axis_index("s") * _NC + lax.axis_index("c")
    b0 = _NBF * wid + jnp.minimum(wid, _REM)
    r0 = b0 * _RPB

    pltpu.sync_copy(src_hbm.at[0, pl.ds(r0, _NBF * _RPB)],
                    idx_v.at[pl.ds(0, _NBF * _RPB)])

    @pl.when(wid < _REM)
    def _():
        pltpu.sync_copy(src_hbm.at[0, pl.ds(r0 + _NBF * _RPB, _RPB)],
                        idx_v.at[pl.ds(_NBF * _RPB, _RPB)])

    def fire_gather(g, p):
        pltpu.make_async_copy(
            x_hbm.at[idx_v.at[pl.ds(g * _GR, _GR)]],
            rows_v.at[p], gsem).start()

    def drain_gather(p):
        pltpu.make_async_copy(x_hbm.at[pl.ds(0, _GR)],
                              rows_v.at[p], gsem).wait()

    def fire_store(g, p):
        pltpu.make_async_copy(
            rows_v.at[p], out_hbm.at[pl.ds(r0 + g * _GR, _GR)],
            wsem).start()

    def drain_store(p):
        pltpu.make_async_copy(
            rows_v.at[p], out_hbm.at[pl.ds(0, _GR)], wsem).wait()

    for p in range(_RING - 1):
        fire_gather(p, p)

    def ring(i, carry):
        gbase = _RING * i
        for j in range(_RING):
            p = j
            pm1 = (j - 1) % _RING
            drain_gather(p)
            if j == 0:
                @pl.when(i > 0)
                def _():
                    drain_store(pm1)
            else:
                drain_store(pm1)
            if j == 0:
                fire_gather(gbase + _RING - 1, pm1)
            else:
                @pl.when(i < _TRIPS - 1)
                def _(j=j, pm1=pm1):
                    fire_gather(gbase + _RING - 1 + j, pm1)
            fire_store(gbase + j, p)
        return carry

    lax.fori_loop(0, _TRIPS, ring, 0)
    drain_store((_G - 1) % _RING)

    @pl.when(wid < _REM)
    def _():
        pltpu.async_copy(
            x_hbm.at[idx_v.at[pl.ds(_NBF * _RPB, _RPB)]],
            rows_v.at[0, pl.ds(0, _RPB)], gsem).wait()
        pltpu.sync_copy(rows_v.at[0, pl.ds(0, _RPB)],
                        out_hbm.at[pl.ds(r0 + _NBF * _RPB, _RPB)])


def kernel(x, edge_index):
    messages = _gather(x, edge_index.astype(jnp.int32))
    return (x, edge_index, messages)
